# baseline (speedup 1.0000x reference)
import functools

import jax
import jax.numpy as jnp
from jax import lax
from jax.experimental import pallas as pl
from jax.experimental.pallas import tpu as pltpu
from jax.experimental.pallas import tpu_sc as plsc

BATCH = 16384
EMBED_DIM = 32
SDAE_DIM = 16
OUT_DIM = EMBED_DIM + SDAE_DIM

_INFO = plsc.get_sparse_core_info()
_NC = _INFO.num_cores
_NS = _INFO.num_subcores
_NW = _NC * _NS
_BPW = BATCH // _NW
_CHUNK = 128
_NCHUNK = _BPW // _CHUNK

_mesh = plsc.VectorSubcoreMesh(core_axis_name="c", subcore_axis_name="s")


@functools.partial(
    pl.kernel,
    mesh=_mesh,
    out_type=jax.ShapeDtypeStruct((BATCH, OUT_DIM), jnp.float32),
    compiler_params=pltpu.CompilerParams(use_tc_tiling_on_sc=False),
    scratch_types=[
        pltpu.VMEM((_NCHUNK, _CHUNK), jnp.int32),
        pltpu.VMEM((_NCHUNK, _CHUNK), jnp.int32),
        pltpu.VMEM((_BPW, EMBED_DIM), jnp.float32),
        pltpu.VMEM((_BPW, EMBED_DIM), jnp.float32),
        pltpu.VMEM((_BPW, SDAE_DIM), jnp.float32),
        pltpu.VMEM((_BPW, SDAE_DIM), jnp.float32),
        pltpu.VMEM((_BPW, OUT_DIM), jnp.float32),
        pltpu.SemaphoreType.DMA,
    ],
)
def _gmf_sc(uidx_hbm, iidx_hbm, usdae_hbm, isdae_hbm, utab_hbm, itab_hbm,
            out_hbm, uidx_v, iidx_v, urows_v, irows_v, usd_v, isd_v, out_v,
            gsem):
    wid = lax.axis_index("s") * _NC + lax.axis_index("c")
    base = wid * _BPW

    pltpu.sync_copy(uidx_hbm.at[pl.ds(wid * _NCHUNK, _NCHUNK), :], uidx_v)
    pltpu.sync_copy(iidx_hbm.at[pl.ds(wid * _NCHUNK, _NCHUNK), :], iidx_v)

    copies = []
    for j in range(_NCHUNK):
        copies.append(pltpu.async_copy(
            utab_hbm.at[uidx_v.at[j]],
            urows_v.at[pl.ds(j * _CHUNK, _CHUNK), :], gsem))
        copies.append(pltpu.async_copy(
            itab_hbm.at[iidx_v.at[j]],
            irows_v.at[pl.ds(j * _CHUNK, _CHUNK), :], gsem))

    pltpu.sync_copy(usdae_hbm.at[pl.ds(base, _BPW), :], usd_v)
    pltpu.sync_copy(isdae_hbm.at[pl.ds(base, _BPW), :], isd_v)

    def sdae_body(r, _):
        out_v[r, pl.ds(EMBED_DIM, SDAE_DIM)] = usd_v[r, :] * isd_v[r, :]
        return _

    lax.fori_loop(0, _BPW, sdae_body, None)

    for c in copies:
        c.wait()

    def embed_body(r, _):
        out_v[r, pl.ds(0, 16)] = urows_v[r, pl.ds(0, 16)] * irows_v[r, pl.ds(0, 16)]
        out_v[r, pl.ds(16, 16)] = urows_v[r, pl.ds(16, 16)] * irows_v[r, pl.ds(16, 16)]
        return _

    lax.fori_loop(0, _BPW, embed_body, None)

    pltpu.sync_copy(out_v, out_hbm.at[pl.ds(base, _BPW), :])


def kernel(user_indices, item_indices, user_sdae_feat, item_sdae_feat,
           user_table, item_table):
    uidx = user_indices.astype(jnp.int32).reshape(_NW * _NCHUNK, _CHUNK)
    iidx = item_indices.astype(jnp.int32).reshape(_NW * _NCHUNK, _CHUNK)
    return _gmf_sc(uidx, iidx, user_sdae_feat, item_sdae_feat,
                   user_table, item_table)
